# baseline (speedup 1.0000x reference)
import jax
import jax.numpy as jnp
from jax import lax
from jax.experimental import pallas as pl
from jax.experimental.pallas import tpu as pltpu
from jax.experimental.pallas import tpu_sc as plsc

TOPK = 2048
TOK_PER_W = 512
CHUNK = 32
NCH = TOK_PER_W // CHUNK


def _stats_kernel(cur_ref, sel_ref, mse_ref):
    L = cur_ref.shape[1]
    C = cur_ref.shape[2]
    TR = 128

    def mean_body(i, acc):
        blk = cur_ref[0, pl.ds(i * TR, TR), :]
        return acc + jnp.sum(blk, axis=0, keepdims=True)

    s = lax.fori_loop(0, L // TR, mean_body, jnp.zeros((1, C), jnp.float32))
    m = s * (1.0 / L)

    def mse_body(i, _):
        blk = cur_ref[0, pl.ds(i * TR, TR), :]
        d = blk - m
        mse_ref[pl.ds(i * TR, TR), :] = jnp.sum(d * d, axis=1, keepdims=True)
        return 0

    lax.fori_loop(0, L // TR, mse_body, 0)

    msef = mse_ref[...].reshape(L // 128, 128)
    bits = lax.bitcast_convert_type(msef, jnp.int32)

    def thresh_body(_, carry):
        lo, hi = carry
        mid = lo + (hi - lo) // 2
        cnt = jnp.sum((bits >= mid).astype(jnp.int32))
        take = cnt >= TOPK
        return (jnp.where(take, mid, lo), jnp.where(take, hi, mid))

    lo, _ = lax.fori_loop(0, 31, thresh_body,
                          (jnp.int32(0), jnp.int32(0x7F800000)))
    thr = lo

    idx = (lax.broadcasted_iota(jnp.int32, (L // 128, 128), 0) * 128
           + lax.broadcasted_iota(jnp.int32, (L // 128, 128), 1))
    n_gt = jnp.sum((bits > thr).astype(jnp.int32))
    need = TOPK - n_gt
    eq = bits == thr

    def cut_body(_, carry):
        lo2, hi2 = carry
        mid = lo2 + (hi2 - lo2) // 2
        g = jnp.sum((eq & (idx < mid)).astype(jnp.int32))
        ok = g >= need
        return (jnp.where(ok, lo2, mid), jnp.where(ok, mid, hi2))

    _, cut = lax.fori_loop(0, 13, cut_body, (jnp.int32(0), jnp.int32(L)))
    cut = jnp.where(need > 0, cut, 0)

    sel = (bits > thr) | (eq & (idx < cut))
    sel_ref[0] = sel.astype(jnp.int32).reshape(L, 1)


def _upmap(tok):
    b = tok >> 12
    l = tok & 4095
    return (b << 10) + ((l >> 7) << 5) + ((l >> 1) & 31)


def _route_body(sel_hbm, cur_hbm, cache_hbm, out_hbm,
                sel_v, rows_a, rows_b, g16_a, g16_b, sem_a, sem_b,
                sem_c, sem_d):
    wid = lax.axis_index("s") * 2 + lax.axis_index("c")
    base = wid * TOK_PER_W

    pltpu.sync_copy(sel_hbm.at[pl.ds(base, TOK_PER_W)], sel_v)

    lane = lax.iota(jnp.int32, 16)

    def p1(c, _):
        def step(buf, isem, nbuf, nsem):
            pltpu.make_async_copy(
                cur_hbm.at[pl.ds(base + c * CHUNK, CHUNK)], buf, isem).wait()

            @pl.when(c + 1 < NCH)
            def _():
                pltpu.make_async_copy(
                    cur_hbm.at[pl.ds(base + (c + 1) * CHUNK, CHUNK)],
                    nbuf, nsem).start()

            cp2 = pltpu.make_async_copy(
                buf, out_hbm.at[pl.ds(base + c * CHUNK, CHUNK)], sem_d)
            cp2.start()
            cp2.wait()

        @pl.when(c % 2 == 0)
        def _():
            step(rows_a, sem_a, rows_b, sem_b)

        @pl.when(c % 2 == 1)
        def _():
            step(rows_b, sem_b, rows_a, sem_a)

        return 0

    pltpu.make_async_copy(
        cur_hbm.at[pl.ds(base, CHUNK)], rows_a, sem_a).start()
    lax.fori_loop(0, NCH, p1, 0)

    def p2(g, _):
        v = sel_v[pl.ds(g * 16, 16)]
        dropped = v == 0
        ones = jnp.where(dropped, jnp.int32(1), jnp.int32(0))
        dc = ones[0]
        pad = jnp.int32(0)
        for i in range(1, 16):
            dc = dc + ones[i]
        for i in range(15, -1, -1):
            pad = jnp.where(v[i] == 0, base + g * 16 + i, pad)
        ids = base + g * 16 + lane
        dst = jnp.where(dropped, ids, pad)
        src = _upmap(dst)

        @pl.when(dc > 0)
        def _():
            cp = pltpu.make_async_copy(cache_hbm.at[src], g16_a, sem_c)
            cp.start()
            cp.wait()
            cp2 = pltpu.make_async_copy(g16_a, out_hbm.at[dst], sem_d)
            cp2.start()
            cp2.wait()

        return 0

    lax.fori_loop(0, TOK_PER_W // 16, p2, 0)


@jax.jit
def kernel(cur_x, cache_x):
    B, L, C = cur_x.shape
    Lc = cache_x.shape[1]

    sel = pl.pallas_call(
        _stats_kernel,
        grid=(B,),
        in_specs=[pl.BlockSpec((1, L, C), lambda b: (b, 0, 0))],
        out_specs=pl.BlockSpec((1, L, 1), lambda b: (b, 0, 0)),
        out_shape=jax.ShapeDtypeStruct((B, L, 1), jnp.int32),
        scratch_shapes=[pltpu.VMEM((L, 1), jnp.float32)],
    )(cur_x)

    sel_flat = sel.reshape(B * L)
    cur_flat = cur_x.reshape(B * L, C)
    cache_flat = cache_x.reshape(B * Lc, C)

    mesh = plsc.VectorSubcoreMesh(core_axis_name="c", subcore_axis_name="s")

    route = pl.kernel(
        _route_body,
        mesh=mesh,
        out_type=jax.ShapeDtypeStruct((B * L, C), jnp.float32),
        scratch_types=[
            pltpu.VMEM((TOK_PER_W,), jnp.int32),
            pltpu.VMEM((CHUNK, 1024), jnp.float32),
            pltpu.VMEM((CHUNK, 1024), jnp.float32),
            pltpu.VMEM((16, 1024), jnp.float32),
            pltpu.VMEM((16, 1024), jnp.float32),
            pltpu.SemaphoreType.DMA,
            pltpu.SemaphoreType.DMA,
            pltpu.SemaphoreType.DMA,
            pltpu.SemaphoreType.DMA,
        ],
    )
    out_flat = route(sel_flat, cur_flat, cache_flat)
    return out_flat.reshape(B, L, C)
